# Initial kernel scaffold; baseline (speedup 1.0000x reference)
import functools

import jax
import jax.numpy as jnp
from jax import lax
from jax.experimental import pallas as pl
from jax.experimental.pallas import tpu as pltpu
from jax.experimental.pallas import tpu_sc as plsc

NA = 50000
NB = 200000
NBR = 6
AF = 133
BF = 147
H = 300
NM = 1000

HP = 304
AP = 50176
BP = 200704
MP = 1024

NC, NS = 2, 16
NW = NC * NS
LN = 16
NV = HP // LN

_mesh = functools.partial(
    plsc.VectorSubcoreMesh, core_axis_name="c", subcore_axis_name="s")


def _wid():
    return lax.axis_index("s") * NC + lax.axis_index("c")


_GS_CA = 16
_GS_ROWS = _GS_CA * NBR
_GS_NCH = (AP // NW) // _GS_CA


def _gather_sum_body(msg_hbm, idx_hbm, out_hbm, idx_v, gbuf, obuf, sem):
    w = _wid()
    pltpu.sync_copy(idx_hbm.at[pl.ds(w * _GS_NCH, _GS_NCH)], idx_v)

    @pl.loop(0, _GS_NCH)
    def _chunk(i):
        pltpu.async_copy(msg_hbm.at[idx_v.at[i]], gbuf, sem).wait()

        @pl.loop(0, _GS_CA)
        def _atom(a):
            r = a * NBR
            for j in range(NV):
                s = pl.ds(j * LN, LN)
                acc = gbuf[r, s]
                for k in range(1, NBR):
                    acc = acc + gbuf[r + k, s]
                obuf[a, s] = acc

        pltpu.sync_copy(obuf, out_hbm.at[pl.ds(w * (AP // NW) + i * _GS_CA,
                                               _GS_CA)])


def _sc_gather_sum(msg, idx_r):
    return pl.kernel(
        _gather_sum_body,
        out_type=jax.ShapeDtypeStruct((AP, HP), jnp.float32),
        mesh=_mesh(),
        scratch_types=[
            pltpu.VMEM((_GS_NCH, _GS_ROWS), jnp.int32),
            pltpu.VMEM((_GS_ROWS, HP), jnp.float32),
            pltpu.VMEM((_GS_CA, HP), jnp.float32),
            pltpu.SemaphoreType.DMA,
        ],
    )(msg, idx_r)


_BU_CB = 64
_BU_NCH = (BP // NW) // _BU_CB


def _bond_update_body(a_hbm, m_hbm, idxa_hbm, idxr_hbm, out_hbm,
                      idxa_v, idxr_v, abuf, mbuf, obuf, sema, semr):
    w = _wid()
    pltpu.sync_copy(idxa_hbm.at[pl.ds(w * _BU_NCH, _BU_NCH)], idxa_v)
    pltpu.sync_copy(idxr_hbm.at[pl.ds(w * _BU_NCH, _BU_NCH)], idxr_v)

    @pl.loop(0, _BU_NCH)
    def _chunk(i):
        cpa = pltpu.async_copy(a_hbm.at[idxa_v.at[i]], abuf, sema)
        cpr = pltpu.async_copy(m_hbm.at[idxr_v.at[i]], mbuf, semr)
        cpa.wait()
        cpr.wait()

        @pl.loop(0, _BU_CB)
        def _row(r):
            for j in range(NV):
                s = pl.ds(j * LN, LN)
                obuf[r, s] = abuf[r, s] - mbuf[r, s]

        pltpu.sync_copy(obuf, out_hbm.at[pl.ds(w * (BP // NW) + i * _BU_CB,
                                               _BU_CB)])


def _sc_bond_update(a_msg, msg, idxa_r, idxr_r):
    return pl.kernel(
        _bond_update_body,
        out_type=jax.ShapeDtypeStruct((BP, HP), jnp.float32),
        mesh=_mesh(),
        scratch_types=[
            pltpu.VMEM((_BU_NCH, _BU_CB), jnp.int32),
            pltpu.VMEM((_BU_NCH, _BU_CB), jnp.int32),
            pltpu.VMEM((_BU_CB, HP), jnp.float32),
            pltpu.VMEM((_BU_CB, HP), jnp.float32),
            pltpu.VMEM((_BU_CB, HP), jnp.float32),
            pltpu.SemaphoreType.DMA,
            pltpu.SemaphoreType.DMA,
        ],
    )(a_msg, msg, idxa_r, idxr_r)


_MR_CA = 56
_MR_NCH = (AP // NW) // _MR_CA
_MR_ZR = MP // NS


def _mol_reduce_body(ah_hbm, ids_hbm, part_hbm, ids_v, rbuf, zbuf, acc_sh):
    w = _wid()
    sid = lax.axis_index("s")
    cid = lax.axis_index("c")

    @pl.loop(0, _MR_ZR)
    def _zrow(r):
        zero = jnp.zeros((LN,), jnp.float32)
        for j in range(NV):
            zbuf[r, pl.ds(j * LN, LN)] = zero

    pltpu.sync_copy(zbuf, acc_sh.at[pl.ds(sid * _MR_ZR, _MR_ZR)])
    plsc.subcore_barrier()

    pltpu.sync_copy(ids_hbm.at[pl.ds(w * _MR_NCH, _MR_NCH)], ids_v)

    @pl.loop(0, _MR_NCH)
    def _chunk(i):
        pltpu.sync_copy(ah_hbm.at[pl.ds(w * (AP // NW) + i * _MR_CA, _MR_CA)],
                        rbuf)
        pltpu.sync_copy(rbuf, acc_sh.at[ids_v.at[i]], add=True)

    plsc.subcore_barrier()

    @pl.when(sid == 0)
    def _emit():
        pltpu.sync_copy(acc_sh, part_hbm.at[cid])


def _sc_mol_reduce(ah, ids_r):
    return pl.kernel(
        _mol_reduce_body,
        out_type=jax.ShapeDtypeStruct((NC, MP, HP), jnp.float32),
        mesh=_mesh(),
        scratch_types=[
            pltpu.VMEM((_MR_NCH, _MR_CA), jnp.int32),
            pltpu.VMEM((_MR_CA, HP), jnp.float32),
            pltpu.VMEM((_MR_ZR, HP), jnp.float32),
            pltpu.VMEM_SHARED((MP, HP), jnp.float32),
        ],
    )(ah, ids_r)


def _mm_in(fb, wi):
    bm = 1024

    def body(x_ref, w_ref, inp_ref, msg_ref):
        acc = jnp.dot(x_ref[...], w_ref[...],
                      preferred_element_type=jnp.float32)
        inp_ref[...] = acc
        msg_ref[...] = jnp.maximum(acc, 0.0)

    return pl.pallas_call(
        body,
        grid=(BP // bm,),
        in_specs=[pl.BlockSpec((bm, BF), lambda i: (i, 0)),
                  pl.BlockSpec((BF, HP), lambda i: (0, 0))],
        out_specs=[pl.BlockSpec((bm, HP), lambda i: (i, 0)),
                   pl.BlockSpec((bm, HP), lambda i: (i, 0))],
        out_shape=[jax.ShapeDtypeStruct((BP, HP), jnp.float32),
                   jax.ShapeDtypeStruct((BP, HP), jnp.float32)],
    )(fb, wi)


def _mm_step(tmp, wh, inp):
    bm = 1024

    def body(x_ref, w_ref, b_ref, o_ref):
        acc = jnp.dot(x_ref[...], w_ref[...],
                      preferred_element_type=jnp.float32)
        o_ref[...] = jnp.maximum(b_ref[...] + acc, 0.0)

    return pl.pallas_call(
        body,
        grid=(BP // bm,),
        in_specs=[pl.BlockSpec((bm, HP), lambda i: (i, 0)),
                  pl.BlockSpec((HP, HP), lambda i: (0, 0)),
                  pl.BlockSpec((bm, HP), lambda i: (i, 0))],
        out_specs=pl.BlockSpec((bm, HP), lambda i: (i, 0)),
        out_shape=jax.ShapeDtypeStruct((BP, HP), jnp.float32),
    )(tmp, wh, inp)


def _mm_out(fa, woa, amsg, woh):
    bm = 1024

    def body(f_ref, wa_ref, a_ref, wh_ref, o_ref):
        acc = jnp.dot(f_ref[...], wa_ref[...],
                      preferred_element_type=jnp.float32)
        acc = acc + jnp.dot(a_ref[...], wh_ref[...],
                            preferred_element_type=jnp.float32)
        acc = jnp.maximum(acc, 0.0)
        col = lax.broadcasted_iota(jnp.int32, (bm, HP), 1)
        o_ref[...] = jnp.where(col == H, 1.0, acc)

    return pl.pallas_call(
        body,
        grid=(AP // bm,),
        in_specs=[pl.BlockSpec((bm, AF), lambda i: (i, 0)),
                  pl.BlockSpec((AF, HP), lambda i: (0, 0)),
                  pl.BlockSpec((bm, HP), lambda i: (i, 0)),
                  pl.BlockSpec((HP, HP), lambda i: (0, 0))],
        out_specs=pl.BlockSpec((bm, HP), lambda i: (i, 0)),
        out_shape=jax.ShapeDtypeStruct((AP, HP), jnp.float32),
    )(fa, woa, amsg, woh)


def _finalize(parts):
    def body(p_ref, o_ref):
        s = p_ref[0, :NM, :] + p_ref[1, :NM, :]
        cnt = jnp.maximum(s[:, H:H + 1], 1.0)
        o_ref[...] = s[:, :H] / cnt

    return pl.pallas_call(
        body,
        out_shape=jax.ShapeDtypeStruct((NM, H), jnp.float32),
    )(parts)


def kernel(f_atoms, f_bonds, a2b, b2a, b2revb, mol_ids, W_i, W_h, W_o):
    f32 = jnp.float32
    fb = jnp.pad(f_bonds.astype(f32), ((0, BP - NB), (0, 0)))
    fa = jnp.pad(f_atoms.astype(f32), ((0, AP - NA), (0, 0)))
    wi = jnp.pad(W_i.astype(f32), ((0, 0), (0, HP - H)))
    wh = jnp.pad(W_h.astype(f32), ((0, HP - H), (0, HP - H)))
    woa = jnp.pad(W_o[:AF].astype(f32), ((0, 0), (0, HP - H)))
    woh = jnp.pad(W_o[AF:].astype(f32), ((0, HP - H), (0, HP - H)))

    i32 = jnp.int32
    a2b_r = jnp.pad(a2b.astype(i32), ((0, AP - NA), (0, 0)))
    a2b_r = a2b_r.reshape(AP * NBR // _GS_ROWS, _GS_ROWS)
    b2a_r = jnp.pad(b2a.astype(i32), (0, BP - NB)).reshape(BP // _BU_CB,
                                                           _BU_CB)
    b2r_r = jnp.pad(b2revb.astype(i32), (0, BP - NB)).reshape(BP // _BU_CB,
                                                              _BU_CB)
    ids_r = jnp.pad(mol_ids.astype(i32), (0, AP - NA),
                    constant_values=NM).reshape(AP // _MR_CA, _MR_CA)

    inp, msg = _mm_in(fb, wi)
    for _ in range(2):
        amsg = _sc_gather_sum(msg, a2b_r)
        tmp = _sc_bond_update(amsg, msg, b2a_r, b2r_r)
        msg = _mm_step(tmp, wh, inp)
    amsg = _sc_gather_sum(msg, a2b_r)
    ah = _mm_out(fa, woa, amsg, woh)
    parts = _sc_mol_reduce(ah, ids_r)
    return _finalize(parts)


# speedup vs baseline: 1.7117x; 1.7117x over previous
import functools

import jax
import jax.numpy as jnp
from jax import lax
from jax.experimental import pallas as pl
from jax.experimental.pallas import tpu as pltpu
from jax.experimental.pallas import tpu_sc as plsc

NA = 50000
NB = 200000
NBR = 6
AF = 133
BF = 147
H = 300
NM = 1000

HP = 384
AP = 50176
BP = 200704
MP = 1024

NC, NS = 2, 16
NW = NC * NS
LN = 16
NV = HP // LN

_mesh = functools.partial(
    plsc.VectorSubcoreMesh, core_axis_name="c", subcore_axis_name="s")


def _wid():
    return lax.axis_index("s") * NC + lax.axis_index("c")


_GS_CA = 16
_GS_ROWS = _GS_CA * NBR
_GS_NCH = (AP // NW) // _GS_CA


def _gather_sum_body(msg_hbm, idx_hbm, out_hbm, idx_v, gbuf, obuf, sem):
    w = _wid()

    @pl.loop(0, _GS_NCH)
    def _chunk(i):
        pltpu.sync_copy(idx_hbm.at[pl.ds((w * _GS_NCH + i) * _GS_ROWS,
                                         _GS_ROWS)], idx_v)
        pltpu.async_copy(msg_hbm.at[idx_v], gbuf, sem).wait()

        @pl.loop(0, _GS_CA)
        def _atom(a):
            r = a * NBR
            for j in range(NV):
                s = pl.ds(j * LN, LN)
                acc = gbuf[r, s]
                for k in range(1, NBR):
                    acc = acc + gbuf[r + k, s]
                obuf[a, s] = acc

        pltpu.sync_copy(obuf, out_hbm.at[pl.ds(w * (AP // NW) + i * _GS_CA,
                                               _GS_CA)])


def _sc_gather_sum(msg, idx_r):
    return pl.kernel(
        _gather_sum_body,
        out_type=jax.ShapeDtypeStruct((AP, HP), jnp.float32),
        mesh=_mesh(),
        scratch_types=[
            pltpu.VMEM((_GS_ROWS,), jnp.int32),
            pltpu.VMEM((_GS_ROWS, HP), jnp.float32),
            pltpu.VMEM((_GS_CA, HP), jnp.float32),
            pltpu.SemaphoreType.DMA,
        ],
    )(msg, idx_r)


_BU_CB = 64
_BU_NCH = (BP // NW) // _BU_CB


def _bond_update_body(a_hbm, m_hbm, idxa_hbm, idxr_hbm, out_hbm,
                      idxa_v, idxr_v, abuf, mbuf, obuf, sema, semr):
    w = _wid()

    @pl.loop(0, _BU_NCH)
    def _chunk(i):
        base = (w * _BU_NCH + i) * _BU_CB
        pltpu.sync_copy(idxa_hbm.at[pl.ds(base, _BU_CB)], idxa_v)
        pltpu.sync_copy(idxr_hbm.at[pl.ds(base, _BU_CB)], idxr_v)
        cpa = pltpu.async_copy(a_hbm.at[idxa_v], abuf, sema)
        cpr = pltpu.async_copy(m_hbm.at[idxr_v], mbuf, semr)
        cpa.wait()
        cpr.wait()

        @pl.loop(0, _BU_CB)
        def _row(r):
            for j in range(NV):
                s = pl.ds(j * LN, LN)
                obuf[r, s] = abuf[r, s] - mbuf[r, s]

        pltpu.sync_copy(obuf, out_hbm.at[pl.ds(w * (BP // NW) + i * _BU_CB,
                                               _BU_CB)])


def _sc_bond_update(a_msg, msg, idxa_r, idxr_r):
    return pl.kernel(
        _bond_update_body,
        out_type=jax.ShapeDtypeStruct((BP, HP), jnp.float32),
        mesh=_mesh(),
        scratch_types=[
            pltpu.VMEM((_BU_CB,), jnp.int32),
            pltpu.VMEM((_BU_CB,), jnp.int32),
            pltpu.VMEM((_BU_CB, HP), jnp.float32),
            pltpu.VMEM((_BU_CB, HP), jnp.float32),
            pltpu.VMEM((_BU_CB, HP), jnp.float32),
            pltpu.SemaphoreType.DMA,
            pltpu.SemaphoreType.DMA,
        ],
    )(a_msg, msg, idxa_r, idxr_r)


def _mol_segsum(ids_r3, ah):
    bm = 1024

    def body(ids_ref, ah_ref, o_ref):
        i = pl.program_id(0)

        @pl.when(i == 0)
        def _init():
            o_ref[...] = jnp.zeros_like(o_ref)

        ids_blk = ids_ref[0, 0, :]
        mol = lax.broadcasted_iota(jnp.int32, (MP, bm), 0)
        oh = (mol == ids_blk[None, :]).astype(jnp.float32)
        o_ref[...] += jnp.dot(oh, ah_ref[...],
                              preferred_element_type=jnp.float32)

    return pl.pallas_call(
        body,
        grid=(AP // bm,),
        in_specs=[pl.BlockSpec((1, 1, bm), lambda i: (i, 0, 0)),
                  pl.BlockSpec((bm, HP), lambda i: (i, 0))],
        out_specs=pl.BlockSpec((MP, HP), lambda i: (0, 0)),
        out_shape=jax.ShapeDtypeStruct((MP, HP), jnp.float32),
    )(ids_r3, ah)


def _mm_in(fb, wi):
    bm = 1024

    def body(x_ref, w_ref, inp_ref, msg_ref):
        acc = jnp.dot(x_ref[...], w_ref[...],
                      preferred_element_type=jnp.float32)
        inp_ref[...] = acc
        msg_ref[...] = jnp.maximum(acc, 0.0)

    return pl.pallas_call(
        body,
        grid=(BP // bm,),
        in_specs=[pl.BlockSpec((bm, BF), lambda i: (i, 0)),
                  pl.BlockSpec((BF, HP), lambda i: (0, 0))],
        out_specs=[pl.BlockSpec((bm, HP), lambda i: (i, 0)),
                   pl.BlockSpec((bm, HP), lambda i: (i, 0))],
        out_shape=[jax.ShapeDtypeStruct((BP, HP), jnp.float32),
                   jax.ShapeDtypeStruct((BP, HP), jnp.float32)],
    )(fb, wi)


def _mm_step(tmp, wh, inp):
    bm = 1024

    def body(x_ref, w_ref, b_ref, o_ref):
        acc = jnp.dot(x_ref[...], w_ref[...],
                      preferred_element_type=jnp.float32)
        o_ref[...] = jnp.maximum(b_ref[...] + acc, 0.0)

    return pl.pallas_call(
        body,
        grid=(BP // bm,),
        in_specs=[pl.BlockSpec((bm, HP), lambda i: (i, 0)),
                  pl.BlockSpec((HP, HP), lambda i: (0, 0)),
                  pl.BlockSpec((bm, HP), lambda i: (i, 0))],
        out_specs=pl.BlockSpec((bm, HP), lambda i: (i, 0)),
        out_shape=jax.ShapeDtypeStruct((BP, HP), jnp.float32),
    )(tmp, wh, inp)


def _mm_out(fa, woa, amsg, woh):
    bm = 1024

    def body(f_ref, wa_ref, a_ref, wh_ref, o_ref):
        acc = jnp.dot(f_ref[...], wa_ref[...],
                      preferred_element_type=jnp.float32)
        acc = acc + jnp.dot(a_ref[...], wh_ref[...],
                            preferred_element_type=jnp.float32)
        acc = jnp.maximum(acc, 0.0)
        col = lax.broadcasted_iota(jnp.int32, (bm, HP), 1)
        o_ref[...] = jnp.where(col == H, 1.0, acc)

    return pl.pallas_call(
        body,
        grid=(AP // bm,),
        in_specs=[pl.BlockSpec((bm, AF), lambda i: (i, 0)),
                  pl.BlockSpec((AF, HP), lambda i: (0, 0)),
                  pl.BlockSpec((bm, HP), lambda i: (i, 0)),
                  pl.BlockSpec((HP, HP), lambda i: (0, 0))],
        out_specs=pl.BlockSpec((bm, HP), lambda i: (i, 0)),
        out_shape=jax.ShapeDtypeStruct((AP, HP), jnp.float32),
    )(fa, woa, amsg, woh)


def _finalize(sums):
    def body(p_ref, o_ref):
        s = p_ref[:NM, :]
        cnt = jnp.maximum(s[:, H:H + 1], 1.0)
        o_ref[...] = s[:, :H] / cnt

    return pl.pallas_call(
        body,
        out_shape=jax.ShapeDtypeStruct((NM, H), jnp.float32),
    )(sums)


def kernel(f_atoms, f_bonds, a2b, b2a, b2revb, mol_ids, W_i, W_h, W_o):
    f32 = jnp.float32
    fb = jnp.pad(f_bonds.astype(f32), ((0, BP - NB), (0, 0)))
    fa = jnp.pad(f_atoms.astype(f32), ((0, AP - NA), (0, 0)))
    wi = jnp.pad(W_i.astype(f32), ((0, 0), (0, HP - H)))
    wh = jnp.pad(W_h.astype(f32), ((0, HP - H), (0, HP - H)))
    woa = jnp.pad(W_o[:AF].astype(f32), ((0, 0), (0, HP - H)))
    woh = jnp.pad(W_o[AF:].astype(f32), ((0, HP - H), (0, HP - H)))

    i32 = jnp.int32
    a2b_r = jnp.pad(a2b.astype(i32), ((0, AP - NA), (0, 0))).reshape(AP * NBR)
    b2a_r = jnp.pad(b2a.astype(i32), (0, BP - NB))
    b2r_r = jnp.pad(b2revb.astype(i32), (0, BP - NB))
    ids_r = jnp.pad(mol_ids.astype(i32), (0, AP - NA),
                    constant_values=NM).reshape(AP // 1024, 1, 1024)

    inp, msg = _mm_in(fb, wi)
    for _ in range(2):
        amsg = _sc_gather_sum(msg, a2b_r)
        tmp = _sc_bond_update(amsg, msg, b2a_r, b2r_r)
        msg = _mm_step(tmp, wh, inp)
    amsg = _sc_gather_sum(msg, a2b_r)
    ah = _mm_out(fa, woa, amsg, woh)
    sums = _mol_segsum(ids_r, ah)
    return _finalize(sums)


# speedup vs baseline: 2.8075x; 1.6402x over previous
import functools

import jax
import jax.numpy as jnp
from jax import lax
from jax.experimental import pallas as pl
from jax.experimental.pallas import tpu as pltpu
from jax.experimental.pallas import tpu_sc as plsc

NA = 50000
NB = 200000
NBR = 6
AF = 133
BF = 147
H = 300
NM = 1000

HP = 384
AP = 50176
BP = 200704
MP = 1024

NC, NS = 2, 16
NW = NC * NS
LN = 16
NV = HP // LN

_mesh = functools.partial(
    plsc.VectorSubcoreMesh, core_axis_name="c", subcore_axis_name="s")


def _wid():
    return lax.axis_index("s") * NC + lax.axis_index("c")


_GS_CA = 16
_GS_ROWS = _GS_CA * NBR
_GS_APW = AP // NW
_GS_NCH = _GS_APW // _GS_CA
_GS_IPW = _GS_APW * NBR


def _gather_sum_body(msg_hbm, idx_hbm, out_hbm, idx_v,
                     gb0, gb1, ob0, ob1, gs0, gs1, ss0, ss1):
    w = _wid()
    pltpu.sync_copy(idx_hbm.at[pl.ds(w * _GS_IPW, _GS_IPW)], idx_v)

    def _gather(c, gb, gs):
        sl = pl.ds(pl.multiple_of(c * _GS_ROWS, _GS_ROWS), _GS_ROWS)
        pltpu.async_copy(msg_hbm.at[idx_v.at[sl]], gb, gs)

    _gather(0, gb0, gs0)
    _gather(1, gb1, gs1)
    rings = ((gb0, gs0, ob0, ss0), (gb1, gs1, ob1, ss1))

    @pl.loop(0, _GS_NCH, step=2)
    def _pair(i):
        for b, (gb, gs, ob, ss) in enumerate(rings):
            cur = i + b
            pltpu.make_async_copy(msg_hbm.at[idx_v.at[pl.ds(0, _GS_ROWS)]],
                                  gb, gs).wait()

            @pl.when(cur >= 2)
            def _drain():
                pltpu.make_async_copy(
                    ob, out_hbm.at[pl.ds(0, _GS_CA)], ss).wait()

            @pl.loop(0, _GS_CA)
            def _atom(a):
                r = a * NBR
                for j in range(NV):
                    s = pl.ds(j * LN, LN)
                    acc = gb[r, s]
                    for k in range(1, NBR):
                        acc = acc + gb[r + k, s]
                    ob[a, s] = acc

            pltpu.async_copy(
                ob, out_hbm.at[pl.ds(w * _GS_APW + cur * _GS_CA, _GS_CA)], ss)

            @pl.when(cur + 2 < _GS_NCH)
            def _next():
                _gather(cur + 2, gb, gs)

    pltpu.make_async_copy(ob0, out_hbm.at[pl.ds(0, _GS_CA)], ss0).wait()
    pltpu.make_async_copy(ob1, out_hbm.at[pl.ds(0, _GS_CA)], ss1).wait()


def _sc_gather_sum(msg, idx_r):
    return pl.kernel(
        _gather_sum_body,
        out_type=jax.ShapeDtypeStruct((AP, HP), jnp.float32),
        mesh=_mesh(),
        scratch_types=[
            pltpu.VMEM((_GS_IPW,), jnp.int32),
            pltpu.VMEM((_GS_ROWS, HP), jnp.float32),
            pltpu.VMEM((_GS_ROWS, HP), jnp.float32),
            pltpu.VMEM((_GS_CA, HP), jnp.float32),
            pltpu.VMEM((_GS_CA, HP), jnp.float32),
            pltpu.SemaphoreType.DMA,
            pltpu.SemaphoreType.DMA,
            pltpu.SemaphoreType.DMA,
            pltpu.SemaphoreType.DMA,
        ],
    )(msg, idx_r)


_BU_CB = 32
_BU_BPW = BP // NW
_BU_NCH = _BU_BPW // _BU_CB


def _bond_update_body(a_hbm, m_hbm, idxa_hbm, idxr_hbm, out_hbm,
                      idxa_v, idxr_v, ab0, ab1, mb0, mb1, ob0, ob1,
                      ga0, ga1, gr0, gr1, ss0, ss1):
    w = _wid()
    pltpu.sync_copy(idxa_hbm.at[pl.ds(w * _BU_BPW, _BU_BPW)], idxa_v)
    pltpu.sync_copy(idxr_hbm.at[pl.ds(w * _BU_BPW, _BU_BPW)], idxr_v)

    def _gather(c, ab, mb, ga, gr):
        sl = pl.ds(pl.multiple_of(c * _BU_CB, _BU_CB), _BU_CB)
        pltpu.async_copy(a_hbm.at[idxa_v.at[sl]], ab, ga)
        pltpu.async_copy(m_hbm.at[idxr_v.at[sl]], mb, gr)

    _gather(0, ab0, mb0, ga0, gr0)
    _gather(1, ab1, mb1, ga1, gr1)
    rings = ((ab0, mb0, ob0, ga0, gr0, ss0), (ab1, mb1, ob1, ga1, gr1, ss1))

    @pl.loop(0, _BU_NCH, step=2)
    def _pair(i):
        for b, (ab, mb, ob, ga, gr, ss) in enumerate(rings):
            cur = i + b
            isl = pl.ds(0, _BU_CB)
            pltpu.make_async_copy(a_hbm.at[idxa_v.at[isl]], ab, ga).wait()
            pltpu.make_async_copy(m_hbm.at[idxr_v.at[isl]], mb, gr).wait()

            @pl.when(cur >= 2)
            def _drain():
                pltpu.make_async_copy(
                    ob, out_hbm.at[pl.ds(0, _BU_CB)], ss).wait()

            @pl.loop(0, _BU_CB)
            def _row(r):
                for j in range(NV):
                    s = pl.ds(j * LN, LN)
                    ob[r, s] = ab[r, s] - mb[r, s]

            pltpu.async_copy(
                ob, out_hbm.at[pl.ds(w * _BU_BPW + cur * _BU_CB, _BU_CB)], ss)

            @pl.when(cur + 2 < _BU_NCH)
            def _next():
                _gather(cur + 2, ab, mb, ga, gr)

    pltpu.make_async_copy(ob0, out_hbm.at[pl.ds(0, _BU_CB)], ss0).wait()
    pltpu.make_async_copy(ob1, out_hbm.at[pl.ds(0, _BU_CB)], ss1).wait()


def _sc_bond_update(a_msg, msg, idxa_r, idxr_r):
    return pl.kernel(
        _bond_update_body,
        out_type=jax.ShapeDtypeStruct((BP, HP), jnp.float32),
        mesh=_mesh(),
        scratch_types=[
            pltpu.VMEM((_BU_BPW,), jnp.int32),
            pltpu.VMEM((_BU_BPW,), jnp.int32),
            pltpu.VMEM((_BU_CB, HP), jnp.float32),
            pltpu.VMEM((_BU_CB, HP), jnp.float32),
            pltpu.VMEM((_BU_CB, HP), jnp.float32),
            pltpu.VMEM((_BU_CB, HP), jnp.float32),
            pltpu.VMEM((_BU_CB, HP), jnp.float32),
            pltpu.VMEM((_BU_CB, HP), jnp.float32),
            pltpu.SemaphoreType.DMA,
            pltpu.SemaphoreType.DMA,
            pltpu.SemaphoreType.DMA,
            pltpu.SemaphoreType.DMA,
            pltpu.SemaphoreType.DMA,
            pltpu.SemaphoreType.DMA,
        ],
    )(a_msg, msg, idxa_r, idxr_r)


def _mol_segsum(ids_r3, ah):
    bm = 1000

    def body(ids_ref, ah_ref, o_ref):
        i = pl.program_id(0)

        @pl.when(i == 0)
        def _init():
            o_ref[...] = jnp.zeros_like(o_ref)

        ids_blk = ids_ref[0, 0, :]
        mol = lax.broadcasted_iota(jnp.int32, (MP, bm), 0)
        oh = (mol == ids_blk[None, :]).astype(jnp.float32)
        o_ref[...] += jnp.dot(oh, ah_ref[...],
                              preferred_element_type=jnp.float32)

    return pl.pallas_call(
        body,
        grid=(NA // bm,),
        in_specs=[pl.BlockSpec((1, 1, bm), lambda i: (i, 0, 0)),
                  pl.BlockSpec((bm, HP), lambda i: (i, 0))],
        out_specs=pl.BlockSpec((MP, HP), lambda i: (0, 0)),
        out_shape=jax.ShapeDtypeStruct((MP, HP), jnp.float32),
    )(ids_r3, ah)


def _mm_in(fb, wi):
    bm = 1000

    def body(x_ref, w_ref, inp_ref, msg_ref):
        acc = jnp.dot(x_ref[...], w_ref[...],
                      preferred_element_type=jnp.float32)
        inp_ref[...] = acc
        msg_ref[...] = jnp.maximum(acc, 0.0)

    return pl.pallas_call(
        body,
        grid=(NB // bm,),
        in_specs=[pl.BlockSpec((bm, BF), lambda i: (i, 0)),
                  pl.BlockSpec((BF, HP), lambda i: (0, 0))],
        out_specs=[pl.BlockSpec((bm, HP), lambda i: (i, 0)),
                   pl.BlockSpec((bm, HP), lambda i: (i, 0))],
        out_shape=[jax.ShapeDtypeStruct((NB, HP), jnp.float32),
                   jax.ShapeDtypeStruct((NB, HP), jnp.float32)],
    )(fb, wi)


def _mm_step(tmp, wh, inp):
    bm = 1000

    def body(x_ref, w_ref, b_ref, o_ref):
        acc = jnp.dot(x_ref[...], w_ref[...],
                      preferred_element_type=jnp.float32)
        o_ref[...] = jnp.maximum(b_ref[...] + acc, 0.0)

    return pl.pallas_call(
        body,
        grid=(NB // bm,),
        in_specs=[pl.BlockSpec((bm, HP), lambda i: (i, 0)),
                  pl.BlockSpec((HP, HP), lambda i: (0, 0)),
                  pl.BlockSpec((bm, HP), lambda i: (i, 0))],
        out_specs=pl.BlockSpec((bm, HP), lambda i: (i, 0)),
        out_shape=jax.ShapeDtypeStruct((NB, HP), jnp.float32),
    )(tmp, wh, inp)


def _mm_out(fa, woa, amsg, woh):
    bm = 1000

    def body(f_ref, wa_ref, a_ref, wh_ref, o_ref):
        acc = jnp.dot(f_ref[...], wa_ref[...],
                      preferred_element_type=jnp.float32)
        acc = acc + jnp.dot(a_ref[...], wh_ref[...],
                            preferred_element_type=jnp.float32)
        acc = jnp.maximum(acc, 0.0)
        col = lax.broadcasted_iota(jnp.int32, (bm, HP), 1)
        o_ref[...] = jnp.where(col == H, 1.0, acc)

    return pl.pallas_call(
        body,
        grid=(NA // bm,),
        in_specs=[pl.BlockSpec((bm, AF), lambda i: (i, 0)),
                  pl.BlockSpec((AF, HP), lambda i: (0, 0)),
                  pl.BlockSpec((bm, HP), lambda i: (i, 0)),
                  pl.BlockSpec((HP, HP), lambda i: (0, 0))],
        out_specs=pl.BlockSpec((bm, HP), lambda i: (i, 0)),
        out_shape=jax.ShapeDtypeStruct((NA, HP), jnp.float32),
    )(fa, woa, amsg, woh)


def _finalize(sums):
    def body(p_ref, o_ref):
        s = p_ref[:NM, :]
        cnt = jnp.maximum(s[:, H:H + 1], 1.0)
        o_ref[...] = s[:, :H] / cnt

    return pl.pallas_call(
        body,
        out_shape=jax.ShapeDtypeStruct((NM, H), jnp.float32),
    )(sums)


def kernel(f_atoms, f_bonds, a2b, b2a, b2revb, mol_ids, W_i, W_h, W_o):
    f32 = jnp.float32
    fb = f_bonds.astype(f32)
    fa = f_atoms.astype(f32)
    wi = jnp.pad(W_i.astype(f32), ((0, 0), (0, HP - H)))
    wh = jnp.pad(W_h.astype(f32), ((0, HP - H), (0, HP - H)))
    woa = jnp.pad(W_o[:AF].astype(f32), ((0, 0), (0, HP - H)))
    woh = jnp.pad(W_o[AF:].astype(f32), ((0, HP - H), (0, HP - H)))

    i32 = jnp.int32
    a2b_r = jnp.pad(a2b.astype(i32), ((0, AP - NA), (0, 0))).reshape(AP * NBR)
    b2a_r = jnp.pad(b2a.astype(i32), (0, BP - NB))
    b2r_r = jnp.pad(b2revb.astype(i32), (0, BP - NB))
    ids_r = mol_ids.astype(i32).reshape(NA // 1000, 1, 1000)

    inp, msg = _mm_in(fb, wi)
    for _ in range(2):
        amsg = _sc_gather_sum(msg, a2b_r)
        tmp = _sc_bond_update(amsg, msg, b2a_r, b2r_r)
        msg = _mm_step(tmp, wh, inp)
    amsg = _sc_gather_sum(msg, a2b_r)
    ah = _mm_out(fa, woa, amsg, woh)
    sums = _mol_segsum(ids_r, ah)
    return _finalize(sums)


# speedup vs baseline: 2.8169x; 1.0033x over previous
import functools

import jax
import jax.numpy as jnp
from jax import lax
from jax.experimental import pallas as pl
from jax.experimental.pallas import tpu as pltpu
from jax.experimental.pallas import tpu_sc as plsc

NA = 50000
NB = 200000
NBR = 6
AF = 133
BF = 147
H = 300
NM = 1000

HP = 384
AP = 50176
BP = 200704
MP = 1024

NC, NS = 2, 16
NW = NC * NS
LN = 16
NV = HP // LN

_mesh = functools.partial(
    plsc.VectorSubcoreMesh, core_axis_name="c", subcore_axis_name="s")


def _wid():
    return lax.axis_index("s") * NC + lax.axis_index("c")


_GS_CA = 16
_GS_ROWS = _GS_CA * NBR
_GS_APW = AP // NW
_GS_NCH = _GS_APW // _GS_CA
_GS_IPW = _GS_APW * NBR


def _gather_sum_body(msg_hbm, idx_hbm, out_hbm, idx_v,
                     gb0, gb1, ob0, ob1, gs0, gs1, ss0, ss1):
    w = _wid()
    pltpu.sync_copy(idx_hbm.at[pl.ds(w * _GS_IPW, _GS_IPW)], idx_v)

    def _gather(c, gb, gs):
        sl = pl.ds(pl.multiple_of(c * _GS_ROWS, _GS_ROWS), _GS_ROWS)
        pltpu.async_copy(msg_hbm.at[idx_v.at[sl]], gb, gs)

    _gather(0, gb0, gs0)
    _gather(1, gb1, gs1)
    rings = ((gb0, gs0, ob0, ss0), (gb1, gs1, ob1, ss1))

    @pl.loop(0, _GS_NCH, step=2)
    def _pair(i):
        for b, (gb, gs, ob, ss) in enumerate(rings):
            cur = i + b
            pltpu.make_async_copy(msg_hbm.at[idx_v.at[pl.ds(0, _GS_ROWS)]],
                                  gb, gs).wait()

            @pl.when(cur >= 2)
            def _drain():
                pltpu.make_async_copy(
                    ob, out_hbm.at[pl.ds(0, _GS_CA)], ss).wait()

            @pl.loop(0, _GS_CA)
            def _atom(a):
                r = a * NBR
                for j in range(NV):
                    s = pl.ds(j * LN, LN)
                    acc = gb[r, s]
                    for k in range(1, NBR):
                        acc = acc + gb[r + k, s]
                    ob[a, s] = acc

            pltpu.async_copy(
                ob, out_hbm.at[pl.ds(w * _GS_APW + cur * _GS_CA, _GS_CA)], ss)

            @pl.when(cur + 2 < _GS_NCH)
            def _next():
                _gather(cur + 2, gb, gs)

    pltpu.make_async_copy(ob0, out_hbm.at[pl.ds(0, _GS_CA)], ss0).wait()
    pltpu.make_async_copy(ob1, out_hbm.at[pl.ds(0, _GS_CA)], ss1).wait()


def _sc_gather_sum(msg, idx_r):
    return pl.kernel(
        _gather_sum_body,
        out_type=jax.ShapeDtypeStruct((AP, HP), jnp.float32),
        mesh=_mesh(),
        scratch_types=[
            pltpu.VMEM((_GS_IPW,), jnp.int32),
            pltpu.VMEM((_GS_ROWS, HP), jnp.float32),
            pltpu.VMEM((_GS_ROWS, HP), jnp.float32),
            pltpu.VMEM((_GS_CA, HP), jnp.float32),
            pltpu.VMEM((_GS_CA, HP), jnp.float32),
            pltpu.SemaphoreType.DMA,
            pltpu.SemaphoreType.DMA,
            pltpu.SemaphoreType.DMA,
            pltpu.SemaphoreType.DMA,
        ],
    )(msg, idx_r)


_BU_CB = 32
_BU_BPW = BP // NW
_BU_NCH = _BU_BPW // _BU_CB


def _bond_update_body(a_hbm, m_hbm, idxa_hbm, idxr_hbm, out_hbm,
                      idxa_v, idxr_v, ab0, ab1, mb0, mb1, ob0, ob1,
                      ga0, ga1, gr0, gr1, ss0, ss1):
    w = _wid()
    pltpu.sync_copy(idxa_hbm.at[pl.ds(w * _BU_BPW, _BU_BPW)], idxa_v)
    pltpu.sync_copy(idxr_hbm.at[pl.ds(w * _BU_BPW, _BU_BPW)], idxr_v)

    def _gather(c, ab, mb, ga, gr):
        sl = pl.ds(pl.multiple_of(c * _BU_CB, _BU_CB), _BU_CB)
        pltpu.async_copy(a_hbm.at[idxa_v.at[sl]], ab, ga)
        pltpu.async_copy(m_hbm.at[idxr_v.at[sl]], mb, gr)

    _gather(0, ab0, mb0, ga0, gr0)
    _gather(1, ab1, mb1, ga1, gr1)
    rings = ((ab0, mb0, ob0, ga0, gr0, ss0), (ab1, mb1, ob1, ga1, gr1, ss1))

    @pl.loop(0, _BU_NCH, step=2)
    def _pair(i):
        for b, (ab, mb, ob, ga, gr, ss) in enumerate(rings):
            cur = i + b
            isl = pl.ds(0, _BU_CB)
            pltpu.make_async_copy(a_hbm.at[idxa_v.at[isl]], ab, ga).wait()
            pltpu.make_async_copy(m_hbm.at[idxr_v.at[isl]], mb, gr).wait()

            @pl.when(cur >= 2)
            def _drain():
                pltpu.make_async_copy(
                    ob, out_hbm.at[pl.ds(0, _BU_CB)], ss).wait()

            @pl.loop(0, _BU_CB)
            def _row(r):
                for j in range(NV):
                    s = pl.ds(j * LN, LN)
                    ob[r, s] = ab[r, s] - mb[r, s]

            pltpu.async_copy(
                ob, out_hbm.at[pl.ds(w * _BU_BPW + cur * _BU_CB, _BU_CB)], ss)

            @pl.when(cur + 2 < _BU_NCH)
            def _next():
                _gather(cur + 2, ab, mb, ga, gr)

    pltpu.make_async_copy(ob0, out_hbm.at[pl.ds(0, _BU_CB)], ss0).wait()
    pltpu.make_async_copy(ob1, out_hbm.at[pl.ds(0, _BU_CB)], ss1).wait()


def _sc_bond_update(a_msg, msg, idxa_r, idxr_r):
    return pl.kernel(
        _bond_update_body,
        out_type=jax.ShapeDtypeStruct((BP, HP), jnp.float32),
        mesh=_mesh(),
        scratch_types=[
            pltpu.VMEM((_BU_BPW,), jnp.int32),
            pltpu.VMEM((_BU_BPW,), jnp.int32),
            pltpu.VMEM((_BU_CB, HP), jnp.float32),
            pltpu.VMEM((_BU_CB, HP), jnp.float32),
            pltpu.VMEM((_BU_CB, HP), jnp.float32),
            pltpu.VMEM((_BU_CB, HP), jnp.float32),
            pltpu.VMEM((_BU_CB, HP), jnp.float32),
            pltpu.VMEM((_BU_CB, HP), jnp.float32),
            pltpu.SemaphoreType.DMA,
            pltpu.SemaphoreType.DMA,
            pltpu.SemaphoreType.DMA,
            pltpu.SemaphoreType.DMA,
            pltpu.SemaphoreType.DMA,
            pltpu.SemaphoreType.DMA,
        ],
    )(a_msg, msg, idxa_r, idxr_r)


def _mol_segsum(ids_r3, ah):
    bm = 1000

    def body(ids_ref, ah_ref, o_ref):
        i = pl.program_id(0)

        @pl.when(i == 0)
        def _init():
            o_ref[...] = jnp.zeros_like(o_ref)

        ids_blk = ids_ref[0, 0, :]
        mol = lax.broadcasted_iota(jnp.int32, (MP, bm), 0)
        oh = (mol == ids_blk[None, :]).astype(jnp.bfloat16)
        o_ref[...] += jnp.dot(oh, ah_ref[...].astype(jnp.bfloat16),
                              preferred_element_type=jnp.float32)

    return pl.pallas_call(
        body,
        grid=(NA // bm,),
        in_specs=[pl.BlockSpec((1, 1, bm), lambda i: (i, 0, 0)),
                  pl.BlockSpec((bm, HP), lambda i: (i, 0))],
        out_specs=pl.BlockSpec((MP, HP), lambda i: (0, 0)),
        out_shape=jax.ShapeDtypeStruct((MP, HP), jnp.float32),
    )(ids_r3, ah)


def _mm_in(fb, wi):
    bm = 1000

    def body(x_ref, w_ref, inp_ref, msg_ref):
        acc = jnp.dot(x_ref[...].astype(jnp.bfloat16), w_ref[...],
                      preferred_element_type=jnp.float32)
        inp_ref[...] = acc
        msg_ref[...] = jnp.maximum(acc, 0.0)

    return pl.pallas_call(
        body,
        grid=(NB // bm,),
        in_specs=[pl.BlockSpec((bm, BF), lambda i: (i, 0)),
                  pl.BlockSpec((BF, HP), lambda i: (0, 0))],
        out_specs=[pl.BlockSpec((bm, HP), lambda i: (i, 0)),
                   pl.BlockSpec((bm, HP), lambda i: (i, 0))],
        out_shape=[jax.ShapeDtypeStruct((NB, HP), jnp.float32),
                   jax.ShapeDtypeStruct((NB, HP), jnp.float32)],
    )(fb, wi)


def _mm_step(tmp, wh, inp):
    bm = 1000

    def body(x_ref, w_ref, b_ref, o_ref):
        acc = jnp.dot(x_ref[...].astype(jnp.bfloat16), w_ref[...],
                      preferred_element_type=jnp.float32)
        o_ref[...] = jnp.maximum(b_ref[...] + acc, 0.0)

    return pl.pallas_call(
        body,
        grid=(NB // bm,),
        in_specs=[pl.BlockSpec((bm, HP), lambda i: (i, 0)),
                  pl.BlockSpec((HP, HP), lambda i: (0, 0)),
                  pl.BlockSpec((bm, HP), lambda i: (i, 0))],
        out_specs=pl.BlockSpec((bm, HP), lambda i: (i, 0)),
        out_shape=jax.ShapeDtypeStruct((NB, HP), jnp.float32),
    )(tmp, wh, inp)


def _mm_out(fa, woa, amsg, woh):
    bm = 1000

    def body(f_ref, wa_ref, a_ref, wh_ref, o_ref):
        acc = jnp.dot(f_ref[...], wa_ref[...],
                      preferred_element_type=jnp.float32)
        acc = acc + jnp.dot(a_ref[...].astype(jnp.bfloat16), wh_ref[...],
                            preferred_element_type=jnp.float32)
        acc = jnp.maximum(acc, 0.0)
        col = lax.broadcasted_iota(jnp.int32, (bm, HP), 1)
        o_ref[...] = jnp.where(col == H, 1.0, acc)

    return pl.pallas_call(
        body,
        grid=(NA // bm,),
        in_specs=[pl.BlockSpec((bm, AF), lambda i: (i, 0)),
                  pl.BlockSpec((AF, HP), lambda i: (0, 0)),
                  pl.BlockSpec((bm, HP), lambda i: (i, 0)),
                  pl.BlockSpec((HP, HP), lambda i: (0, 0))],
        out_specs=pl.BlockSpec((bm, HP), lambda i: (i, 0)),
        out_shape=jax.ShapeDtypeStruct((NA, HP), jnp.float32),
    )(fa, woa, amsg, woh)


def _finalize(sums):
    def body(p_ref, o_ref):
        s = p_ref[:NM, :]
        cnt = jnp.maximum(s[:, H:H + 1], 1.0)
        o_ref[...] = s[:, :H] / cnt

    return pl.pallas_call(
        body,
        out_shape=jax.ShapeDtypeStruct((NM, H), jnp.float32),
    )(sums)


def kernel(f_atoms, f_bonds, a2b, b2a, b2revb, mol_ids, W_i, W_h, W_o):
    bf16 = jnp.bfloat16
    fb = f_bonds
    fa = f_atoms.astype(bf16)
    wi = jnp.pad(W_i.astype(bf16), ((0, 0), (0, HP - H)))
    wh = jnp.pad(W_h.astype(bf16), ((0, HP - H), (0, HP - H)))
    woa = jnp.pad(W_o[:AF].astype(bf16), ((0, 0), (0, HP - H)))
    woh = jnp.pad(W_o[AF:].astype(bf16), ((0, HP - H), (0, HP - H)))

    i32 = jnp.int32
    a2b_r = jnp.pad(a2b.astype(i32), ((0, AP - NA), (0, 0))).reshape(AP * NBR)
    b2a_r = jnp.pad(b2a.astype(i32), (0, BP - NB))
    b2r_r = jnp.pad(b2revb.astype(i32), (0, BP - NB))
    ids_r = mol_ids.astype(i32).reshape(NA // 1000, 1, 1000)

    inp, msg = _mm_in(fb, wi)
    for _ in range(2):
        amsg = _sc_gather_sum(msg, a2b_r)
        tmp = _sc_bond_update(amsg, msg, b2a_r, b2r_r)
        msg = _mm_step(tmp, wh, inp)
    amsg = _sc_gather_sum(msg, a2b_r)
    ah = _mm_out(fa, woa, amsg, woh)
    sums = _mol_segsum(ids_r, ah)
    return _finalize(sums)


# speedup vs baseline: 2.8472x; 1.0107x over previous
import functools

import jax
import jax.numpy as jnp
from jax import lax
from jax.experimental import pallas as pl
from jax.experimental.pallas import tpu as pltpu
from jax.experimental.pallas import tpu_sc as plsc

NA = 50000
NB = 200000
NBR = 6
AF = 133
BF = 147
H = 300
NM = 1000

HP = 384
AP = 50176
BP = 200704
MP = 1024

NC, NS = 2, 16
NW = NC * NS
LN = 16
NV = HP // LN

_mesh = functools.partial(
    plsc.VectorSubcoreMesh, core_axis_name="c", subcore_axis_name="s")


def _wid():
    return lax.axis_index("s") * NC + lax.axis_index("c")


_GS_CA = 8
_GS_ROWS = _GS_CA * NBR
_GS_APW = AP // NW
_GS_NCH = _GS_APW // _GS_CA
_GS_IPW = _GS_APW * NBR
_GS_NR = 4


def _gather_sum_body(msg_hbm, idx_hbm, out_hbm, idx_v,
                     gb0, gb1, gb2, gb3, ob0, ob1, ob2, ob3,
                     gs0, gs1, gs2, gs3, ss0, ss1, ss2, ss3):
    w = _wid()
    pltpu.sync_copy(idx_hbm.at[pl.ds(w * _GS_IPW, _GS_IPW)], idx_v)

    def _gather(c, gb, gs):
        sl = pl.ds(pl.multiple_of(c * _GS_ROWS, _GS_ROWS), _GS_ROWS)
        pltpu.async_copy(msg_hbm.at[idx_v.at[sl]], gb, gs)

    rings = ((gb0, gs0, ob0, ss0), (gb1, gs1, ob1, ss1),
             (gb2, gs2, ob2, ss2), (gb3, gs3, ob3, ss3))
    for b, (gb, gs, _, _) in enumerate(rings):
        _gather(b, gb, gs)

    @pl.loop(0, _GS_NCH, step=_GS_NR)
    def _grp(i):
        for b, (gb, gs, ob, ss) in enumerate(rings):
            cur = i + b
            pltpu.make_async_copy(msg_hbm.at[idx_v.at[pl.ds(0, _GS_ROWS)]],
                                  gb, gs).wait()

            @pl.when(cur >= _GS_NR)
            def _drain():
                pltpu.make_async_copy(
                    ob, out_hbm.at[pl.ds(0, _GS_CA)], ss).wait()

            @pl.loop(0, _GS_CA)
            def _atom(a):
                r = a * NBR
                for j in range(NV):
                    s = pl.ds(j * LN, LN)
                    acc = gb[r, s]
                    for k in range(1, NBR):
                        acc = acc + gb[r + k, s]
                    ob[a, s] = acc

            pltpu.async_copy(
                ob, out_hbm.at[pl.ds(w * _GS_APW + cur * _GS_CA, _GS_CA)], ss)

            @pl.when(cur + _GS_NR < _GS_NCH)
            def _next():
                _gather(cur + _GS_NR, gb, gs)

    for _, (_, _, ob, ss) in enumerate(rings):
        pltpu.make_async_copy(ob, out_hbm.at[pl.ds(0, _GS_CA)], ss).wait()


def _sc_gather_sum(msg, idx_r):
    return pl.kernel(
        _gather_sum_body,
        out_type=jax.ShapeDtypeStruct((AP, HP), jnp.float32),
        mesh=_mesh(),
        scratch_types=(
            [pltpu.VMEM((_GS_IPW,), jnp.int32)]
            + [pltpu.VMEM((_GS_ROWS, HP), jnp.float32)] * _GS_NR
            + [pltpu.VMEM((_GS_CA, HP), jnp.float32)] * _GS_NR
            + [pltpu.SemaphoreType.DMA] * (2 * _GS_NR)
        ),
    )(msg, idx_r)


_BU_CB = 32
_BU_BPW = BP // NW
_BU_NCH = _BU_BPW // _BU_CB


def _bond_update_body(a_hbm, m_hbm, idxa_hbm, idxr_hbm, out_hbm,
                      idxa_v, idxr_v, ab0, ab1, mb0, mb1, ob0, ob1,
                      ga0, ga1, gr0, gr1, ss0, ss1):
    w = _wid()
    pltpu.sync_copy(idxa_hbm.at[pl.ds(w * _BU_BPW, _BU_BPW)], idxa_v)
    pltpu.sync_copy(idxr_hbm.at[pl.ds(w * _BU_BPW, _BU_BPW)], idxr_v)

    def _gather(c, ab, mb, ga, gr):
        sl = pl.ds(pl.multiple_of(c * _BU_CB, _BU_CB), _BU_CB)
        pltpu.async_copy(a_hbm.at[idxa_v.at[sl]], ab, ga)
        pltpu.async_copy(m_hbm.at[idxr_v.at[sl]], mb, gr)

    _gather(0, ab0, mb0, ga0, gr0)
    _gather(1, ab1, mb1, ga1, gr1)
    rings = ((ab0, mb0, ob0, ga0, gr0, ss0), (ab1, mb1, ob1, ga1, gr1, ss1))

    @pl.loop(0, _BU_NCH, step=2)
    def _pair(i):
        for b, (ab, mb, ob, ga, gr, ss) in enumerate(rings):
            cur = i + b
            isl = pl.ds(0, _BU_CB)
            pltpu.make_async_copy(a_hbm.at[idxa_v.at[isl]], ab, ga).wait()
            pltpu.make_async_copy(m_hbm.at[idxr_v.at[isl]], mb, gr).wait()

            @pl.when(cur >= 2)
            def _drain():
                pltpu.make_async_copy(
                    ob, out_hbm.at[pl.ds(0, _BU_CB)], ss).wait()

            @pl.loop(0, _BU_CB)
            def _row(r):
                for j in range(NV):
                    s = pl.ds(j * LN, LN)
                    ob[r, s] = ab[r, s] - mb[r, s]

            pltpu.async_copy(
                ob, out_hbm.at[pl.ds(w * _BU_BPW + cur * _BU_CB, _BU_CB)], ss)

            @pl.when(cur + 2 < _BU_NCH)
            def _next():
                _gather(cur + 2, ab, mb, ga, gr)

    pltpu.make_async_copy(ob0, out_hbm.at[pl.ds(0, _BU_CB)], ss0).wait()
    pltpu.make_async_copy(ob1, out_hbm.at[pl.ds(0, _BU_CB)], ss1).wait()


def _sc_bond_update(a_msg, msg, idxa_r, idxr_r):
    return pl.kernel(
        _bond_update_body,
        out_type=jax.ShapeDtypeStruct((BP, HP), jnp.float32),
        mesh=_mesh(),
        scratch_types=[
            pltpu.VMEM((_BU_BPW,), jnp.int32),
            pltpu.VMEM((_BU_BPW,), jnp.int32),
            pltpu.VMEM((_BU_CB, HP), jnp.float32),
            pltpu.VMEM((_BU_CB, HP), jnp.float32),
            pltpu.VMEM((_BU_CB, HP), jnp.float32),
            pltpu.VMEM((_BU_CB, HP), jnp.float32),
            pltpu.VMEM((_BU_CB, HP), jnp.float32),
            pltpu.VMEM((_BU_CB, HP), jnp.float32),
            pltpu.SemaphoreType.DMA,
            pltpu.SemaphoreType.DMA,
            pltpu.SemaphoreType.DMA,
            pltpu.SemaphoreType.DMA,
            pltpu.SemaphoreType.DMA,
            pltpu.SemaphoreType.DMA,
        ],
    )(a_msg, msg, idxa_r, idxr_r)


def _mol_segsum(ids_r3, ah):
    bm = 1000

    def body(ids_ref, ah_ref, o_ref):
        i = pl.program_id(0)

        @pl.when(i == 0)
        def _init():
            o_ref[...] = jnp.zeros_like(o_ref)

        ids_blk = ids_ref[0, 0, :]
        mol = lax.broadcasted_iota(jnp.int32, (MP, bm), 0)
        oh = (mol == ids_blk[None, :]).astype(jnp.bfloat16)
        o_ref[...] += jnp.dot(oh, ah_ref[...].astype(jnp.bfloat16),
                              preferred_element_type=jnp.float32)

    return pl.pallas_call(
        body,
        grid=(NA // bm,),
        in_specs=[pl.BlockSpec((1, 1, bm), lambda i: (i, 0, 0)),
                  pl.BlockSpec((bm, HP), lambda i: (i, 0))],
        out_specs=pl.BlockSpec((MP, HP), lambda i: (0, 0)),
        out_shape=jax.ShapeDtypeStruct((MP, HP), jnp.float32),
    )(ids_r3, ah)


def _mm_in(fb, wi):
    bm = 1000

    def body(x_ref, w_ref, inp_ref, msg_ref):
        acc = jnp.dot(x_ref[...], w_ref[...],
                      preferred_element_type=jnp.float32)
        inp_ref[...] = acc.astype(jnp.bfloat16)
        msg_ref[...] = jnp.maximum(acc, 0.0)

    return pl.pallas_call(
        body,
        grid=(NB // bm,),
        in_specs=[pl.BlockSpec((bm, BF), lambda i: (i, 0)),
                  pl.BlockSpec((BF, HP), lambda i: (0, 0))],
        out_specs=[pl.BlockSpec((bm, HP), lambda i: (i, 0)),
                   pl.BlockSpec((bm, HP), lambda i: (i, 0))],
        out_shape=[jax.ShapeDtypeStruct((NB, HP), jnp.bfloat16),
                   jax.ShapeDtypeStruct((NB, HP), jnp.float32)],
    )(fb, wi)


def _mm_step(tmp, wh, inp):
    bm = 1000

    def body(x_ref, w_ref, b_ref, o_ref):
        acc = jnp.dot(x_ref[...].astype(jnp.bfloat16), w_ref[...],
                      preferred_element_type=jnp.float32)
        o_ref[...] = jnp.maximum(b_ref[...].astype(jnp.float32) + acc, 0.0)

    return pl.pallas_call(
        body,
        grid=(NB // bm,),
        in_specs=[pl.BlockSpec((bm, HP), lambda i: (i, 0)),
                  pl.BlockSpec((HP, HP), lambda i: (0, 0)),
                  pl.BlockSpec((bm, HP), lambda i: (i, 0))],
        out_specs=pl.BlockSpec((bm, HP), lambda i: (i, 0)),
        out_shape=jax.ShapeDtypeStruct((NB, HP), jnp.float32),
    )(tmp, wh, inp)


def _mm_out(fa, woa, amsg, woh):
    bm = 1000

    def body(f_ref, wa_ref, a_ref, wh_ref, o_ref):
        acc = jnp.dot(f_ref[...], wa_ref[...],
                      preferred_element_type=jnp.float32)
        acc = acc + jnp.dot(a_ref[...].astype(jnp.bfloat16), wh_ref[...],
                            preferred_element_type=jnp.float32)
        acc = jnp.maximum(acc, 0.0)
        col = lax.broadcasted_iota(jnp.int32, (bm, HP), 1)
        o_ref[...] = jnp.where(col == H, 1.0, acc)

    return pl.pallas_call(
        body,
        grid=(NA // bm,),
        in_specs=[pl.BlockSpec((bm, AF), lambda i: (i, 0)),
                  pl.BlockSpec((AF, HP), lambda i: (0, 0)),
                  pl.BlockSpec((bm, HP), lambda i: (i, 0)),
                  pl.BlockSpec((HP, HP), lambda i: (0, 0))],
        out_specs=pl.BlockSpec((bm, HP), lambda i: (i, 0)),
        out_shape=jax.ShapeDtypeStruct((NA, HP), jnp.float32),
    )(fa, woa, amsg, woh)


def _finalize(sums):
    def body(p_ref, o_ref):
        s = p_ref[:NM, :]
        cnt = jnp.maximum(s[:, H:H + 1], 1.0)
        o_ref[...] = s[:, :H] / cnt

    return pl.pallas_call(
        body,
        out_shape=jax.ShapeDtypeStruct((NM, H), jnp.float32),
    )(sums)


def kernel(f_atoms, f_bonds, a2b, b2a, b2revb, mol_ids, W_i, W_h, W_o):
    bf16 = jnp.bfloat16
    fb = f_bonds.astype(bf16)
    fa = f_atoms.astype(bf16)
    wi = jnp.pad(W_i.astype(bf16), ((0, 0), (0, HP - H)))
    wh = jnp.pad(W_h.astype(bf16), ((0, HP - H), (0, HP - H)))
    woa = jnp.pad(W_o[:AF].astype(bf16), ((0, 0), (0, HP - H)))
    woh = jnp.pad(W_o[AF:].astype(bf16), ((0, HP - H), (0, HP - H)))

    i32 = jnp.int32
    a2b_r = jnp.pad(a2b.astype(i32), ((0, AP - NA), (0, 0))).reshape(AP * NBR)
    b2a_r = jnp.pad(b2a.astype(i32), (0, BP - NB))
    b2r_r = jnp.pad(b2revb.astype(i32), (0, BP - NB))
    ids_r = mol_ids.astype(i32).reshape(NA // 1000, 1, 1000)

    inp, msg = _mm_in(fb, wi)
    for _ in range(2):
        amsg = _sc_gather_sum(msg, a2b_r)
        tmp = _sc_bond_update(amsg, msg, b2a_r, b2r_r)
        msg = _mm_step(tmp, wh, inp)
    amsg = _sc_gather_sum(msg, a2b_r)
    ah = _mm_out(fa, woa, amsg, woh)
    sums = _mol_segsum(ids_r, ah)
    return _finalize(sums)


# speedup vs baseline: 3.0342x; 1.0657x over previous
import functools

import jax
import jax.numpy as jnp
from jax import lax
from jax.experimental import pallas as pl
from jax.experimental.pallas import tpu as pltpu
from jax.experimental.pallas import tpu_sc as plsc

NA = 50000
NB = 200000
NBR = 6
AF = 133
BF = 147
H = 300
NM = 1000

HP = 384
AP = 50176
BP = 200704
MP = 1024

NC, NS = 2, 16
NW = NC * NS
LN = 16
NV = HP // LN

_mesh = functools.partial(
    plsc.VectorSubcoreMesh, core_axis_name="c", subcore_axis_name="s")


def _wid():
    return lax.axis_index("s") * NC + lax.axis_index("c")


_GS_CA = 16
_GS_HR = 48
_GS_ROWS = _GS_CA * NBR
_GS_APW = AP // NW
_GS_NCH = _GS_APW // _GS_CA
_GS_IPW = _GS_APW * NBR


def _gather_sum_body(msg_hbm, idx_hbm, out_hbm, idx_v,
                     ga0, gb0, ga1, gb1, ob0, ob1,
                     sa0, sb0, sa1, sb1, ss0, ss1):
    w = _wid()
    pltpu.sync_copy(idx_hbm.at[pl.ds(w * _GS_IPW, _GS_IPW)], idx_v)

    def _gather(c, ga, gb, sa, sb):
        base = pl.multiple_of(c * _GS_ROWS, _GS_ROWS)
        pltpu.async_copy(msg_hbm.at[idx_v.at[pl.ds(base, _GS_HR)]], ga, sa)
        base2 = pl.multiple_of(c * _GS_ROWS + _GS_HR, _GS_HR)
        pltpu.async_copy(msg_hbm.at[idx_v.at[pl.ds(base2, _GS_HR)]], gb, sb)

    _gather(0, ga0, gb0, sa0, sb0)
    _gather(1, ga1, gb1, sa1, sb1)
    rings = ((ga0, gb0, ob0, sa0, sb0, ss0), (ga1, gb1, ob1, sa1, sb1, ss1))

    @pl.loop(0, _GS_NCH, step=2)
    def _pair(i):
        for b, (ga, gb, ob, sa, sb, ss) in enumerate(rings):
            cur = i + b
            isl = pl.ds(0, _GS_HR)
            pltpu.make_async_copy(msg_hbm.at[idx_v.at[isl]], ga, sa).wait()
            pltpu.make_async_copy(msg_hbm.at[idx_v.at[isl]], gb, sb).wait()

            @pl.when(cur >= 2)
            def _drain():
                pltpu.make_async_copy(
                    ob, out_hbm.at[pl.ds(0, _GS_CA)], ss).wait()

            @pl.loop(0, 8)
            def _atom_a(a):
                r = a * NBR
                for j in range(NV):
                    sj = pl.ds(j * LN, LN)
                    acc = ga[r, sj]
                    for k in range(1, NBR):
                        acc = acc + ga[r + k, sj]
                    ob[a, sj] = acc

            @pl.loop(0, 8)
            def _atom_b(a):
                r = a * NBR
                for j in range(NV):
                    sj = pl.ds(j * LN, LN)
                    acc = gb[r, sj]
                    for k in range(1, NBR):
                        acc = acc + gb[r + k, sj]
                    ob[a + 8, sj] = acc

            pltpu.async_copy(
                ob, out_hbm.at[pl.ds(w * _GS_APW + cur * _GS_CA, _GS_CA)], ss)

            @pl.when(cur + 2 < _GS_NCH)
            def _next():
                _gather(cur + 2, ga, gb, sa, sb)

    pltpu.make_async_copy(ob0, out_hbm.at[pl.ds(0, _GS_CA)], ss0).wait()
    pltpu.make_async_copy(ob1, out_hbm.at[pl.ds(0, _GS_CA)], ss1).wait()


def _sc_gather_sum(msg, idx_r):
    return pl.kernel(
        _gather_sum_body,
        out_type=jax.ShapeDtypeStruct((AP, HP), jnp.float32),
        mesh=_mesh(),
        scratch_types=(
            [pltpu.VMEM((_GS_IPW,), jnp.int32)]
            + [pltpu.VMEM((_GS_HR, HP), jnp.float32)] * 4
            + [pltpu.VMEM((_GS_CA, HP), jnp.float32)] * 2
            + [pltpu.SemaphoreType.DMA] * 6
        ),
    )(msg, idx_r)


_BU_CB = 32
_BU_BPW = BP // NW
_BU_NCH = _BU_BPW // _BU_CB


def _bond_update_body(a_hbm, m_hbm, idxa_hbm, idxr_hbm, out_hbm,
                      idxa_v, idxr_v, ab0, ab1, mb0, mb1, ob0, ob1,
                      ga0, ga1, gr0, gr1, ss0, ss1):
    w = _wid()
    pltpu.sync_copy(idxa_hbm.at[pl.ds(w * _BU_BPW, _BU_BPW)], idxa_v)
    pltpu.sync_copy(idxr_hbm.at[pl.ds(w * _BU_BPW, _BU_BPW)], idxr_v)

    def _gather(c, ab, mb, ga, gr):
        sl = pl.ds(pl.multiple_of(c * _BU_CB, _BU_CB), _BU_CB)
        pltpu.async_copy(a_hbm.at[idxa_v.at[sl]], ab, ga)
        pltpu.async_copy(m_hbm.at[idxr_v.at[sl]], mb, gr)

    _gather(0, ab0, mb0, ga0, gr0)
    _gather(1, ab1, mb1, ga1, gr1)
    rings = ((ab0, mb0, ob0, ga0, gr0, ss0), (ab1, mb1, ob1, ga1, gr1, ss1))

    @pl.loop(0, _BU_NCH, step=2)
    def _pair(i):
        for b, (ab, mb, ob, ga, gr, ss) in enumerate(rings):
            cur = i + b
            isl = pl.ds(0, _BU_CB)
            pltpu.make_async_copy(a_hbm.at[idxa_v.at[isl]], ab, ga).wait()
            pltpu.make_async_copy(m_hbm.at[idxr_v.at[isl]], mb, gr).wait()

            @pl.when(cur >= 2)
            def _drain():
                pltpu.make_async_copy(
                    ob, out_hbm.at[pl.ds(0, _BU_CB)], ss).wait()

            @pl.loop(0, _BU_CB)
            def _row(r):
                for j in range(NV):
                    s = pl.ds(j * LN, LN)
                    ob[r, s] = ab[r, s] - mb[r, s]

            pltpu.async_copy(
                ob, out_hbm.at[pl.ds(w * _BU_BPW + cur * _BU_CB, _BU_CB)], ss)

            @pl.when(cur + 2 < _BU_NCH)
            def _next():
                _gather(cur + 2, ab, mb, ga, gr)

    pltpu.make_async_copy(ob0, out_hbm.at[pl.ds(0, _BU_CB)], ss0).wait()
    pltpu.make_async_copy(ob1, out_hbm.at[pl.ds(0, _BU_CB)], ss1).wait()


def _sc_bond_update(a_msg, msg, idxa_r, idxr_r):
    return pl.kernel(
        _bond_update_body,
        out_type=jax.ShapeDtypeStruct((BP, HP), jnp.float32),
        mesh=_mesh(),
        scratch_types=[
            pltpu.VMEM((_BU_BPW,), jnp.int32),
            pltpu.VMEM((_BU_BPW,), jnp.int32),
            pltpu.VMEM((_BU_CB, HP), jnp.float32),
            pltpu.VMEM((_BU_CB, HP), jnp.float32),
            pltpu.VMEM((_BU_CB, HP), jnp.float32),
            pltpu.VMEM((_BU_CB, HP), jnp.float32),
            pltpu.VMEM((_BU_CB, HP), jnp.float32),
            pltpu.VMEM((_BU_CB, HP), jnp.float32),
            pltpu.SemaphoreType.DMA,
            pltpu.SemaphoreType.DMA,
            pltpu.SemaphoreType.DMA,
            pltpu.SemaphoreType.DMA,
            pltpu.SemaphoreType.DMA,
            pltpu.SemaphoreType.DMA,
        ],
    )(a_msg, msg, idxa_r, idxr_r)


def _mol_segsum(ids_r3, ah):
    bm = 1000

    def body(ids_ref, ah_ref, o_ref):
        i = pl.program_id(0)

        @pl.when(i == 0)
        def _init():
            o_ref[...] = jnp.zeros_like(o_ref)

        ids_blk = ids_ref[0, 0, :]
        mol = lax.broadcasted_iota(jnp.int32, (MP, bm), 0)
        oh = (mol == ids_blk[None, :]).astype(jnp.bfloat16)
        o_ref[...] += jnp.dot(oh, ah_ref[...].astype(jnp.bfloat16),
                              preferred_element_type=jnp.float32)

    return pl.pallas_call(
        body,
        grid=(NA // bm,),
        in_specs=[pl.BlockSpec((1, 1, bm), lambda i: (i, 0, 0)),
                  pl.BlockSpec((bm, HP), lambda i: (i, 0))],
        out_specs=pl.BlockSpec((MP, HP), lambda i: (0, 0)),
        out_shape=jax.ShapeDtypeStruct((MP, HP), jnp.float32),
    )(ids_r3, ah)


def _mm_in(fb, wi):
    bm = 2000

    def body(x_ref, w_ref, inp_ref, msg_ref):
        acc = jnp.dot(x_ref[...], w_ref[...],
                      preferred_element_type=jnp.float32)
        inp_ref[...] = acc.astype(jnp.bfloat16)
        msg_ref[...] = jnp.maximum(acc, 0.0)

    return pl.pallas_call(
        body,
        grid=(NB // bm,),
        in_specs=[pl.BlockSpec((bm, BF), lambda i: (i, 0)),
                  pl.BlockSpec((BF, HP), lambda i: (0, 0))],
        out_specs=[pl.BlockSpec((bm, HP), lambda i: (i, 0)),
                   pl.BlockSpec((bm, HP), lambda i: (i, 0))],
        out_shape=[jax.ShapeDtypeStruct((NB, HP), jnp.bfloat16),
                   jax.ShapeDtypeStruct((NB, HP), jnp.float32)],
    )(fb, wi)


def _mm_step(tmp, wh, inp):
    bm = 2000

    def body(x_ref, w_ref, b_ref, o_ref):
        acc = jnp.dot(x_ref[...].astype(jnp.bfloat16), w_ref[...],
                      preferred_element_type=jnp.float32)
        o_ref[...] = jnp.maximum(b_ref[...].astype(jnp.float32) + acc, 0.0)

    return pl.pallas_call(
        body,
        grid=(NB // bm,),
        in_specs=[pl.BlockSpec((bm, HP), lambda i: (i, 0)),
                  pl.BlockSpec((HP, HP), lambda i: (0, 0)),
                  pl.BlockSpec((bm, HP), lambda i: (i, 0))],
        out_specs=pl.BlockSpec((bm, HP), lambda i: (i, 0)),
        out_shape=jax.ShapeDtypeStruct((NB, HP), jnp.float32),
    )(tmp, wh, inp)


def _mm_out(fa, woa, amsg, woh):
    bm = 1000

    def body(f_ref, wa_ref, a_ref, wh_ref, o_ref):
        acc = jnp.dot(f_ref[...], wa_ref[...],
                      preferred_element_type=jnp.float32)
        acc = acc + jnp.dot(a_ref[...].astype(jnp.bfloat16), wh_ref[...],
                            preferred_element_type=jnp.float32)
        acc = jnp.maximum(acc, 0.0)
        col = lax.broadcasted_iota(jnp.int32, (bm, HP), 1)
        o_ref[...] = jnp.where(col == H, 1.0, acc)

    return pl.pallas_call(
        body,
        grid=(NA // bm,),
        in_specs=[pl.BlockSpec((bm, AF), lambda i: (i, 0)),
                  pl.BlockSpec((AF, HP), lambda i: (0, 0)),
                  pl.BlockSpec((bm, HP), lambda i: (i, 0)),
                  pl.BlockSpec((HP, HP), lambda i: (0, 0))],
        out_specs=pl.BlockSpec((bm, HP), lambda i: (i, 0)),
        out_shape=jax.ShapeDtypeStruct((NA, HP), jnp.float32),
    )(fa, woa, amsg, woh)


def _finalize(sums):
    def body(p_ref, o_ref):
        s = p_ref[:NM, :]
        cnt = jnp.maximum(s[:, H:H + 1], 1.0)
        o_ref[...] = s[:, :H] / cnt

    return pl.pallas_call(
        body,
        out_shape=jax.ShapeDtypeStruct((NM, H), jnp.float32),
    )(sums)


def kernel(f_atoms, f_bonds, a2b, b2a, b2revb, mol_ids, W_i, W_h, W_o):
    bf16 = jnp.bfloat16
    fb = f_bonds.astype(bf16)
    fa = f_atoms.astype(bf16)
    wi = jnp.pad(W_i.astype(bf16), ((0, 0), (0, HP - H)))
    wh = jnp.pad(W_h.astype(bf16), ((0, HP - H), (0, HP - H)))
    woa = jnp.pad(W_o[:AF].astype(bf16), ((0, 0), (0, HP - H)))
    woh = jnp.pad(W_o[AF:].astype(bf16), ((0, HP - H), (0, HP - H)))

    i32 = jnp.int32
    a2b_r = jnp.pad(a2b.astype(i32), ((0, AP - NA), (0, 0))).reshape(AP * NBR)
    b2a_r = jnp.pad(b2a.astype(i32), (0, BP - NB))
    b2r_r = jnp.pad(b2revb.astype(i32), (0, BP - NB))
    ids_r = mol_ids.astype(i32).reshape(NA // 1000, 1, 1000)

    inp, msg = _mm_in(fb, wi)
    for _ in range(2):
        amsg = _sc_gather_sum(msg, a2b_r)
        tmp = _sc_bond_update(amsg, msg, b2a_r, b2r_r)
        msg = _mm_step(tmp, wh, inp)
    amsg = _sc_gather_sum(msg, a2b_r)
    ah = _mm_out(fa, woa, amsg, woh)
    sums = _mol_segsum(ids_r, ah)
    return _finalize(sums)


# speedup vs baseline: 3.2027x; 1.0556x over previous
import functools

import jax
import jax.numpy as jnp
from jax import lax
from jax.experimental import pallas as pl
from jax.experimental.pallas import tpu as pltpu
from jax.experimental.pallas import tpu_sc as plsc

NA = 50000
NB = 200000
NBR = 6
AF = 133
BF = 147
H = 300
NM = 1000

HP = 384
AP = 50176
BP = 200704
MP = 1024

NC, NS = 2, 16
NW = NC * NS
LN = 16
NV = HP // LN

_mesh = functools.partial(
    plsc.VectorSubcoreMesh, core_axis_name="c", subcore_axis_name="s")


def _wid():
    return lax.axis_index("s") * NC + lax.axis_index("c")


_GS_CA = 16
_GS_HR = 48
_GS_ROWS = _GS_CA * NBR
_GS_APW = AP // NW
_GS_NCH = _GS_APW // _GS_CA
_GS_IPW = _GS_APW * NBR


def _gather_sum_body(msg_hbm, idx_hbm, out_hbm, idx_v,
                     ga0, gb0, ga1, gb1, ob0, ob1,
                     sa0, sb0, sa1, sb1, ss0, ss1):
    w = _wid()
    pltpu.sync_copy(idx_hbm.at[pl.ds(w * _GS_IPW, _GS_IPW)], idx_v)

    def _gather(c, ga, gb, sa, sb):
        base = pl.multiple_of(c * _GS_ROWS, _GS_ROWS)
        pltpu.async_copy(msg_hbm.at[idx_v.at[pl.ds(base, _GS_HR)]], ga, sa)
        base2 = pl.multiple_of(c * _GS_ROWS + _GS_HR, _GS_HR)
        pltpu.async_copy(msg_hbm.at[idx_v.at[pl.ds(base2, _GS_HR)]], gb, sb)

    _gather(0, ga0, gb0, sa0, sb0)
    _gather(1, ga1, gb1, sa1, sb1)
    rings = ((ga0, gb0, ob0, sa0, sb0, ss0), (ga1, gb1, ob1, sa1, sb1, ss1))

    @pl.loop(0, _GS_NCH, step=2)
    def _pair(i):
        for b, (ga, gb, ob, sa, sb, ss) in enumerate(rings):
            cur = i + b
            isl = pl.ds(0, _GS_HR)
            pltpu.make_async_copy(msg_hbm.at[idx_v.at[isl]], ga, sa).wait()
            pltpu.make_async_copy(msg_hbm.at[idx_v.at[isl]], gb, sb).wait()

            @pl.when(cur >= 2)
            def _drain():
                pltpu.make_async_copy(
                    ob, out_hbm.at[pl.ds(0, _GS_CA)], ss).wait()

            @pl.loop(0, 8)
            def _atom_a(a):
                r = a * NBR
                for j in range(NV):
                    sj = pl.ds(j * LN, LN)
                    acc = ga[r, sj]
                    for k in range(1, NBR):
                        acc = acc + ga[r + k, sj]
                    ob[a, sj] = acc

            @pl.loop(0, 8)
            def _atom_b(a):
                r = a * NBR
                for j in range(NV):
                    sj = pl.ds(j * LN, LN)
                    acc = gb[r, sj]
                    for k in range(1, NBR):
                        acc = acc + gb[r + k, sj]
                    ob[a + 8, sj] = acc

            pltpu.async_copy(
                ob, out_hbm.at[pl.ds(w * _GS_APW + cur * _GS_CA, _GS_CA)], ss)

            @pl.when(cur + 2 < _GS_NCH)
            def _next():
                _gather(cur + 2, ga, gb, sa, sb)

    pltpu.make_async_copy(ob0, out_hbm.at[pl.ds(0, _GS_CA)], ss0).wait()
    pltpu.make_async_copy(ob1, out_hbm.at[pl.ds(0, _GS_CA)], ss1).wait()


def _sc_gather_sum(msg, idx_r):
    return pl.kernel(
        _gather_sum_body,
        out_type=jax.ShapeDtypeStruct((AP, HP), jnp.float32),
        mesh=_mesh(),
        scratch_types=(
            [pltpu.VMEM((_GS_IPW,), jnp.int32)]
            + [pltpu.VMEM((_GS_HR, HP), jnp.float32)] * 4
            + [pltpu.VMEM((_GS_CA, HP), jnp.float32)] * 2
            + [pltpu.SemaphoreType.DMA] * 6
        ),
    )(msg, idx_r)


_BU_CB = 32
_BU_BPW = BP // NW
_BU_NCH = _BU_BPW // _BU_CB


def _bond_update_body(a_hbm, m_hbm, idxa_hbm, idxr_hbm, out_hbm,
                      idxa_v, idxr_v, ab0, ab1, mb0, mb1, ob0, ob1,
                      ga0, ga1, gr0, gr1, ss0, ss1):
    w = _wid()
    pltpu.sync_copy(idxa_hbm.at[pl.ds(w * _BU_BPW, _BU_BPW)], idxa_v)
    pltpu.sync_copy(idxr_hbm.at[pl.ds(w * _BU_BPW, _BU_BPW)], idxr_v)

    def _gather(c, ab, mb, ga, gr):
        sl = pl.ds(pl.multiple_of(c * _BU_CB, _BU_CB), _BU_CB)
        pltpu.async_copy(a_hbm.at[idxa_v.at[sl]], ab, ga)
        pltpu.async_copy(m_hbm.at[idxr_v.at[sl]], mb, gr)

    _gather(0, ab0, mb0, ga0, gr0)
    _gather(1, ab1, mb1, ga1, gr1)
    rings = ((ab0, mb0, ob0, ga0, gr0, ss0), (ab1, mb1, ob1, ga1, gr1, ss1))

    @pl.loop(0, _BU_NCH, step=2)
    def _pair(i):
        for b, (ab, mb, ob, ga, gr, ss) in enumerate(rings):
            cur = i + b
            isl = pl.ds(0, _BU_CB)
            pltpu.make_async_copy(a_hbm.at[idxa_v.at[isl]], ab, ga).wait()
            pltpu.make_async_copy(m_hbm.at[idxr_v.at[isl]], mb, gr).wait()

            @pl.when(cur >= 2)
            def _drain():
                pltpu.make_async_copy(
                    ob, out_hbm.at[pl.ds(0, _BU_CB)], ss).wait()

            @pl.loop(0, _BU_CB)
            def _row(r):
                for j in range(NV):
                    s = pl.ds(j * LN, LN)
                    ob[r, s] = ab[r, s] - mb[r, s]

            pltpu.async_copy(
                ob, out_hbm.at[pl.ds(w * _BU_BPW + cur * _BU_CB, _BU_CB)], ss)

            @pl.when(cur + 2 < _BU_NCH)
            def _next():
                _gather(cur + 2, ab, mb, ga, gr)

    pltpu.make_async_copy(ob0, out_hbm.at[pl.ds(0, _BU_CB)], ss0).wait()
    pltpu.make_async_copy(ob1, out_hbm.at[pl.ds(0, _BU_CB)], ss1).wait()


def _sc_bond_update(a_msg, msg, idxa_r, idxr_r):
    return pl.kernel(
        _bond_update_body,
        out_type=jax.ShapeDtypeStruct((BP, HP), jnp.float32),
        mesh=_mesh(),
        scratch_types=[
            pltpu.VMEM((_BU_BPW,), jnp.int32),
            pltpu.VMEM((_BU_BPW,), jnp.int32),
            pltpu.VMEM((_BU_CB, HP), jnp.float32),
            pltpu.VMEM((_BU_CB, HP), jnp.float32),
            pltpu.VMEM((_BU_CB, HP), jnp.float32),
            pltpu.VMEM((_BU_CB, HP), jnp.float32),
            pltpu.VMEM((_BU_CB, HP), jnp.float32),
            pltpu.VMEM((_BU_CB, HP), jnp.float32),
            pltpu.SemaphoreType.DMA,
            pltpu.SemaphoreType.DMA,
            pltpu.SemaphoreType.DMA,
            pltpu.SemaphoreType.DMA,
            pltpu.SemaphoreType.DMA,
            pltpu.SemaphoreType.DMA,
        ],
    )(a_msg, msg, idxa_r, idxr_r)


def _mol_segsum(ids_r3, ah):
    bm = 1000

    def body(ids_ref, ah_ref, o_ref):
        i = pl.program_id(0)

        @pl.when(i == 0)
        def _init():
            o_ref[...] = jnp.zeros_like(o_ref)

        ids_blk = ids_ref[0, 0, :]
        mol = lax.broadcasted_iota(jnp.int32, (MP, bm), 0)
        oh = (mol == ids_blk[None, :]).astype(jnp.bfloat16)
        o_ref[...] += jnp.dot(oh, ah_ref[...].astype(jnp.bfloat16),
                              preferred_element_type=jnp.float32)

    return pl.pallas_call(
        body,
        grid=(NA // bm,),
        in_specs=[pl.BlockSpec((1, 1, bm), lambda i: (i, 0, 0)),
                  pl.BlockSpec((bm, HP), lambda i: (i, 0))],
        out_specs=pl.BlockSpec((MP, HP), lambda i: (0, 0)),
        out_shape=jax.ShapeDtypeStruct((MP, HP), jnp.float32),
    )(ids_r3, ah)


def _mm_in(fbt, wi):
    bm = 2048

    def body(x_ref, w_ref, inp_ref, msg_ref):
        acc = lax.dot_general(x_ref[...], w_ref[...],
                              (((0,), (0,)), ((), ())),
                              preferred_element_type=jnp.float32)
        inp_ref[...] = acc.astype(jnp.bfloat16)
        msg_ref[...] = jnp.maximum(acc, 0.0)

    return pl.pallas_call(
        body,
        grid=(pl.cdiv(NB, bm),),
        in_specs=[pl.BlockSpec((BF, bm), lambda i: (0, i)),
                  pl.BlockSpec((BF, HP), lambda i: (0, 0))],
        out_specs=[pl.BlockSpec((bm, HP), lambda i: (i, 0)),
                   pl.BlockSpec((bm, HP), lambda i: (i, 0))],
        out_shape=[jax.ShapeDtypeStruct((NB, HP), jnp.bfloat16),
                   jax.ShapeDtypeStruct((NB, HP), jnp.float32)],
    )(fbt, wi)


def _mm_step(tmp, wh, inp):
    bm = 2000

    def body(x_ref, w_ref, b_ref, o_ref):
        acc = jnp.dot(x_ref[...].astype(jnp.bfloat16), w_ref[...],
                      preferred_element_type=jnp.float32)
        o_ref[...] = jnp.maximum(b_ref[...].astype(jnp.float32) + acc, 0.0)

    return pl.pallas_call(
        body,
        grid=(NB // bm,),
        in_specs=[pl.BlockSpec((bm, HP), lambda i: (i, 0)),
                  pl.BlockSpec((HP, HP), lambda i: (0, 0)),
                  pl.BlockSpec((bm, HP), lambda i: (i, 0))],
        out_specs=pl.BlockSpec((bm, HP), lambda i: (i, 0)),
        out_shape=jax.ShapeDtypeStruct((NB, HP), jnp.float32),
    )(tmp, wh, inp)


def _mm_out(fat, woa, amsg, woh):
    bm = 1024

    def body(f_ref, wa_ref, a_ref, wh_ref, o_ref):
        acc = lax.dot_general(f_ref[...], wa_ref[...],
                              (((0,), (0,)), ((), ())),
                              preferred_element_type=jnp.float32)
        acc = acc + jnp.dot(a_ref[...].astype(jnp.bfloat16), wh_ref[...],
                            preferred_element_type=jnp.float32)
        acc = jnp.maximum(acc, 0.0)
        col = lax.broadcasted_iota(jnp.int32, (bm, HP), 1)
        o_ref[...] = jnp.where(col == H, 1.0, acc)

    return pl.pallas_call(
        body,
        grid=(pl.cdiv(NA, bm),),
        in_specs=[pl.BlockSpec((AF, bm), lambda i: (0, i)),
                  pl.BlockSpec((AF, HP), lambda i: (0, 0)),
                  pl.BlockSpec((bm, HP), lambda i: (i, 0)),
                  pl.BlockSpec((HP, HP), lambda i: (0, 0))],
        out_specs=pl.BlockSpec((bm, HP), lambda i: (i, 0)),
        out_shape=jax.ShapeDtypeStruct((NA, HP), jnp.float32),
    )(fat, woa, amsg, woh)


def _finalize(sums):
    def body(p_ref, o_ref):
        s = p_ref[:NM, :]
        cnt = jnp.maximum(s[:, H:H + 1], 1.0)
        o_ref[...] = s[:, :H] / cnt

    return pl.pallas_call(
        body,
        out_shape=jax.ShapeDtypeStruct((NM, H), jnp.float32),
    )(sums)


def kernel(f_atoms, f_bonds, a2b, b2a, b2revb, mol_ids, W_i, W_h, W_o):
    bf16 = jnp.bfloat16
    fb = f_bonds.astype(bf16).T
    fa = f_atoms.astype(bf16).T
    wi = jnp.pad(W_i.astype(bf16), ((0, 0), (0, HP - H)))
    wh = jnp.pad(W_h.astype(bf16), ((0, HP - H), (0, HP - H)))
    woa = jnp.pad(W_o[:AF].astype(bf16), ((0, 0), (0, HP - H)))
    woh = jnp.pad(W_o[AF:].astype(bf16), ((0, HP - H), (0, HP - H)))

    i32 = jnp.int32
    a2b_r = jnp.pad(a2b.astype(i32), ((0, AP - NA), (0, 0))).reshape(AP * NBR)
    b2a_r = jnp.pad(b2a.astype(i32), (0, BP - NB))
    b2r_r = jnp.pad(b2revb.astype(i32), (0, BP - NB))
    ids_r = mol_ids.astype(i32).reshape(NA // 1000, 1, 1000)

    inp, msg = _mm_in(fb, wi)
    for _ in range(2):
        amsg = _sc_gather_sum(msg, a2b_r)
        tmp = _sc_bond_update(amsg, msg, b2a_r, b2r_r)
        msg = _mm_step(tmp, wh, inp)
    amsg = _sc_gather_sum(msg, a2b_r)
    ah = _mm_out(fa, woa, amsg, woh)
    sums = _mol_segsum(ids_r, ah)
    return _finalize(sums)
